# baseline (speedup 1.0000x reference)
import functools

import jax
import jax.numpy as jnp
from jax import lax
from jax.experimental import pallas as pl
from jax.experimental.pallas import tpu as pltpu
from jax.experimental.pallas import tpu_sc as plsc

ROWS = 16384
COLS = 200
NW = 32
ROWS_W = ROWS // NW
CHUNK_R = 128
NCHUNK = ROWS_W // CHUNK_R
LANES = 16
ROWS_PER_IT = 4


def _sc_kernel(x_hbm, t_hbm, out_hbm, xb0, xb1, tbuf, pairbuf, accbuf,
               sem0, sem1):
    wid = lax.axis_index("s") * 2 + lax.axis_index("c")
    base = wid * ROWS_W

    lane = lax.iota(jnp.int32, 16)
    head8 = lane < 8

    pltpu.sync_copy(t_hbm, tbuf)
    valid = lane < 10
    row_ids = jnp.where(valid, lane, 0)
    rowsum = jnp.zeros((LANES,), jnp.float32)
    for c in range(3):
        col_ids = jnp.full((LANES,), c, jnp.int32)
        rowsum = rowsum + plsc.load_gather(tbuf, [row_ids, col_ids])
    rowsum = jnp.where(valid, rowsum, 0.0)
    accbuf[...] = rowsum
    lookup = accbuf

    for b in range(11):
        pairbuf[pl.ds(11 * b, LANES)] = rowsum + rowsum[b]

    bufs = (xb0, xb1)
    sems = (sem0, sem1)

    def chunk_body(xb):
        def body(i, accs):
            accs = list(accs)
            for rr in range(ROWS_PER_IT):
                r = i * ROWS_PER_IT + rr
                for k in range(6):
                    i0 = xb[r, pl.ds(k * 32, LANES)]
                    i1 = xb[r, pl.ds(k * 32 + 16, LANES)]
                    comb = i0 + i1 * 11
                    g = plsc.load_gather(pairbuf, [comb])
                    accs[k] = accs[k] + g
                tail = xb[r, pl.ds(184, LANES)]
                tail = jnp.where(head8, 10, tail)
                g = plsc.load_gather(lookup, [tail])
                accs[rr % 6] = accs[rr % 6] + g
            return tuple(accs)
        return body

    zero = jnp.zeros((LANES,), jnp.float32)
    accs = (zero,) * 6

    copies = [None] * NCHUNK
    copies[0] = pltpu.async_copy(
        x_hbm.at[pl.ds(base, CHUNK_R)], bufs[0], sems[0])
    for c in range(NCHUNK):
        copies[c].wait()
        if c + 1 < NCHUNK:
            copies[c + 1] = pltpu.async_copy(
                x_hbm.at[pl.ds(base + (c + 1) * CHUNK_R, CHUNK_R)],
                bufs[(c + 1) % 2], sems[(c + 1) % 2])
        accs = lax.fori_loop(0, CHUNK_R // ROWS_PER_IT,
                             chunk_body(bufs[c % 2]), accs)

    accbuf[...] = ((accs[0] + accs[1]) + (accs[2] + accs[3])
                   + (accs[4] + accs[5]))
    pltpu.sync_copy(accbuf, out_hbm.at[wid])


@jax.jit
def kernel(x, table):
    k = functools.partial(
        pl.kernel,
        mesh=plsc.VectorSubcoreMesh(core_axis_name="c", subcore_axis_name="s"),
        out_type=jax.ShapeDtypeStruct((NW, LANES), jnp.float32),
        compiler_params=pltpu.CompilerParams(needs_layout_passes=False),
        scratch_types=[
            pltpu.VMEM((CHUNK_R, COLS), jnp.int32),
            pltpu.VMEM((CHUNK_R, COLS), jnp.int32),
            pltpu.VMEM((10, 3), jnp.float32),
            pltpu.VMEM((11 * 11 + 5, ), jnp.float32),
            pltpu.VMEM((LANES,), jnp.float32),
            pltpu.SemaphoreType.DMA,
            pltpu.SemaphoreType.DMA,
        ],
    )(_sc_kernel)
    partials = k(x, table)
    return partials.sum()


# speedup vs baseline: 1.0039x; 1.0039x over previous
import functools

import jax
import jax.numpy as jnp
from jax import lax
from jax.experimental import pallas as pl
from jax.experimental.pallas import tpu as pltpu
from jax.experimental.pallas import tpu_sc as plsc

ROWS = 16384
COLS = 200
SC_ROWS = 12288
TC_ROWS = ROWS - SC_ROWS
NW = 32
ROWS_W = SC_ROWS // NW
NCHUNK = 4
CHUNK_R = ROWS_W // NCHUNK
LANES = 16
ROWS_PER_IT = 2
TC_BLOCK_R = 512


def _sc_kernel(x_hbm, t_hbm, out_hbm, xb0, xb1, tbuf, pairbuf, accbuf,
               sem0, sem1):
    wid = lax.axis_index("s") * 2 + lax.axis_index("c")
    base = wid * ROWS_W

    lane = lax.iota(jnp.int32, 16)
    head8 = lane < 8

    pltpu.sync_copy(t_hbm, tbuf)
    valid = lane < 10
    row_ids = jnp.where(valid, lane, 0)
    rowsum = jnp.zeros((LANES,), jnp.float32)
    for c in range(3):
        col_ids = jnp.full((LANES,), c, jnp.int32)
        rowsum = rowsum + plsc.load_gather(tbuf, [row_ids, col_ids])
    rowsum = jnp.where(valid, rowsum, 0.0)
    accbuf[...] = rowsum
    lookup = accbuf

    for b in range(11):
        pairbuf[pl.ds(11 * b, LANES)] = rowsum + rowsum[b]

    bufs = (xb0, xb1)
    sems = (sem0, sem1)

    def chunk_body(xb):
        def body(i, accs):
            accs = list(accs)
            for rr in range(ROWS_PER_IT):
                r = i * ROWS_PER_IT + rr
                for k in range(6):
                    i0 = xb[r, pl.ds(k * 32, LANES)]
                    i1 = xb[r, pl.ds(k * 32 + 16, LANES)]
                    comb = i0 + i1 * 11
                    g = plsc.load_gather(pairbuf, [comb])
                    accs[k % 4] = accs[k % 4] + g
                tail = xb[r, pl.ds(184, LANES)]
                tail = jnp.where(head8, 10, tail)
                g = plsc.load_gather(lookup, [tail])
                accs[rr] = accs[rr] + g
            return tuple(accs)
        return body

    zero = jnp.zeros((LANES,), jnp.float32)
    accs = (zero, zero, zero, zero)

    copies = [None] * NCHUNK
    copies[0] = pltpu.async_copy(
        x_hbm.at[pl.ds(base, CHUNK_R)], bufs[0], sems[0])
    for c in range(NCHUNK):
        copies[c].wait()
        if c + 1 < NCHUNK:
            copies[c + 1] = pltpu.async_copy(
                x_hbm.at[pl.ds(base + (c + 1) * CHUNK_R, CHUNK_R)],
                bufs[(c + 1) % 2], sems[(c + 1) % 2])
        accs = lax.fori_loop(0, CHUNK_R // ROWS_PER_IT,
                             chunk_body(bufs[c % 2]), accs)

    accbuf[...] = (accs[0] + accs[1]) + (accs[2] + accs[3])
    pltpu.sync_copy(accbuf, out_hbm.at[wid])


def _tc_kernel(t_ref, x_ref, o_ref):
    i = pl.program_id(0)

    @pl.when(i == 0)
    def _():
        o_ref[...] = jnp.zeros_like(o_ref)

    xb = x_ref[...]
    rs = jnp.sum(t_ref[...], axis=1)
    val = jnp.zeros(xb.shape, jnp.float32)
    for v in range(10):
        val = val + jnp.where(xb == v, rs[v], 0.0)
    o_ref[...] = o_ref[...] + jnp.sum(val, axis=0, keepdims=True)


@jax.jit
def kernel(x, table):
    k = functools.partial(
        pl.kernel,
        mesh=plsc.VectorSubcoreMesh(core_axis_name="c", subcore_axis_name="s"),
        out_type=jax.ShapeDtypeStruct((NW, LANES), jnp.float32),
        compiler_params=pltpu.CompilerParams(needs_layout_passes=False),
        scratch_types=[
            pltpu.VMEM((CHUNK_R, COLS), jnp.int32),
            pltpu.VMEM((CHUNK_R, COLS), jnp.int32),
            pltpu.VMEM((10, 3), jnp.float32),
            pltpu.VMEM((11 * 11 + 5, ), jnp.float32),
            pltpu.VMEM((LANES,), jnp.float32),
            pltpu.SemaphoreType.DMA,
            pltpu.SemaphoreType.DMA,
        ],
    )(_sc_kernel)
    partials = k(x, table)

    tc_sum = pl.pallas_call(
        _tc_kernel,
        grid=(TC_ROWS // TC_BLOCK_R,),
        in_specs=[
            pl.BlockSpec((10, 3), lambda i: (0, 0)),
            pl.BlockSpec((TC_BLOCK_R, COLS),
                         lambda i: (SC_ROWS // TC_BLOCK_R + i, 0)),
        ],
        out_specs=pl.BlockSpec((1, COLS), lambda i: (0, 0)),
        out_shape=jax.ShapeDtypeStruct((1, COLS), jnp.float32),
    )(table, x)

    return partials.sum() + tc_sum.sum()


# speedup vs baseline: 1.0125x; 1.0086x over previous
import functools

import jax
import jax.numpy as jnp
from jax import lax
from jax.experimental import pallas as pl
from jax.experimental.pallas import tpu as pltpu
from jax.experimental.pallas import tpu_sc as plsc

ROWS = 16384
COLS = 200
NW = 32
ROWS_W = ROWS // NW
CHUNK_R = 128
NCHUNK = ROWS_W // CHUNK_R
LANES = 16
ROWS_PER_IT = 2


def _sc_kernel(x_hbm, t_hbm, out_hbm, xb0, xb1, tbuf, pairbuf, accbuf,
               sem0, sem1):
    wid = lax.axis_index("s") * 2 + lax.axis_index("c")
    base = wid * ROWS_W

    lane = lax.iota(jnp.int32, 16)
    head8 = lane < 8

    pltpu.sync_copy(t_hbm, tbuf)
    valid = lane < 10
    row_ids = jnp.where(valid, lane, 0)
    rowsum = jnp.zeros((LANES,), jnp.float32)
    for c in range(3):
        col_ids = jnp.full((LANES,), c, jnp.int32)
        rowsum = rowsum + plsc.load_gather(tbuf, [row_ids, col_ids])
    rowsum = jnp.where(valid, rowsum, 0.0)
    accbuf[...] = rowsum
    lookup = accbuf

    for b in range(11):
        pairbuf[pl.ds(11 * b, LANES)] = rowsum + rowsum[b]

    bufs = (xb0, xb1)
    sems = (sem0, sem1)

    def chunk_body(xb):
        def body(i, accs):
            accs = list(accs)
            for rr in range(ROWS_PER_IT):
                r = i * ROWS_PER_IT + rr
                for k in range(6):
                    i0 = xb[r, pl.ds(k * 32, LANES)]
                    i1 = xb[r, pl.ds(k * 32 + 16, LANES)]
                    comb = i0 + i1 * 11
                    g = plsc.load_gather(pairbuf, [comb])
                    accs[k % 4] = accs[k % 4] + g
                tail = xb[r, pl.ds(184, LANES)]
                tail = jnp.where(head8, 10, tail)
                g = plsc.load_gather(lookup, [tail])
                accs[rr] = accs[rr] + g
            return tuple(accs)
        return body

    zero = jnp.zeros((LANES,), jnp.float32)
    accs = (zero, zero, zero, zero)

    copies = [None] * NCHUNK
    copies[0] = pltpu.async_copy(
        x_hbm.at[pl.ds(base, CHUNK_R)], bufs[0], sems[0])
    for c in range(NCHUNK):
        copies[c].wait()
        if c + 1 < NCHUNK:
            copies[c + 1] = pltpu.async_copy(
                x_hbm.at[pl.ds(base + (c + 1) * CHUNK_R, CHUNK_R)],
                bufs[(c + 1) % 2], sems[(c + 1) % 2])
        accs = lax.fori_loop(0, CHUNK_R // ROWS_PER_IT,
                             chunk_body(bufs[c % 2]), accs)

    accbuf[...] = (accs[0] + accs[1]) + (accs[2] + accs[3])
    pltpu.sync_copy(accbuf, out_hbm.at[wid])


@jax.jit
def kernel(x, table):
    k = functools.partial(
        pl.kernel,
        mesh=plsc.VectorSubcoreMesh(core_axis_name="c", subcore_axis_name="s"),
        out_type=jax.ShapeDtypeStruct((NW, LANES), jnp.float32),
        compiler_params=pltpu.CompilerParams(needs_layout_passes=False),
        scratch_types=[
            pltpu.VMEM((CHUNK_R, COLS), jnp.int32),
            pltpu.VMEM((CHUNK_R, COLS), jnp.int32),
            pltpu.VMEM((10, 3), jnp.float32),
            pltpu.VMEM((11 * 11 + 5, ), jnp.float32),
            pltpu.VMEM((LANES,), jnp.float32),
            pltpu.SemaphoreType.DMA,
            pltpu.SemaphoreType.DMA,
        ],
    )(_sc_kernel)
    partials = k(x, table)
    return partials.sum()
